# baseline (speedup 1.0000x reference)
import functools

import jax
import jax.numpy as jnp
from jax import lax
from jax.experimental import pallas as pl
from jax.experimental.pallas import tpu as pltpu
from jax.experimental.pallas import tpu_sc as plsc

K = 8192
D = 256
N = 16384
COMMITMENT_COST = 0.25

TILE = 256
GRID = N // TILE


GROUPS = ((0, 2736), (2736, 5472), (5472, 8192))


def _argmin_body(x_ref, w_ref, x2_ref, w2_ref, idx_ref, mind_ref):
    x = x_ref[...]
    w = w_ref[...]
    mm = lax.dot_general(x.astype(jnp.bfloat16), w.astype(jnp.bfloat16),
                         (((1,), (1,)), ((), ())),
                         preferred_element_type=jnp.float32)
    d = (x2_ref[...] + w2_ref[...]) - 2.0 * mm
    iota = lax.broadcasted_iota(jnp.int32, (TILE, K), 1)
    inf = jnp.float32(jnp.inf)
    ms, js = [], []
    for lo, hi in GROUPS:
        mask = (iota >= lo) & (iota < hi)
        dg = jnp.where(mask, d, inf)
        mg = jnp.min(dg, axis=1, keepdims=True)
        jg = jnp.min(jnp.where(dg == mg, iota, K), axis=1, keepdims=True)
        ms.append(mg)
        js.append(jg)
    a_v, a_i = ms[0], js[0]
    for g in (1, 2):
        a_vb = a_v.astype(jnp.bfloat16).astype(jnp.float32)
        m_g, i_g = ms[g], js[g]
        keep_v = a_vb < m_g
        eq = a_vb == m_g
        a_v = jnp.where(keep_v, a_vb, m_g)
        a_i = jnp.where(keep_v | (eq & (a_i < i_g)), a_i, i_g)
    idx_ref[0, 0, :] = a_i[:, 0]
    mind_ref[0, 0, :] = jnp.minimum(jnp.minimum(ms[0], ms[1]), ms[2])[:, 0]


def _nearest(x2d, W, x2, w2):
    return pl.pallas_call(
        _argmin_body,
        grid=(GRID,),
        in_specs=[
            pl.BlockSpec((TILE, D), lambda i: (i, 0)),
            pl.BlockSpec((K, D), lambda i: (0, 0)),
            pl.BlockSpec((TILE, 1), lambda i: (i, 0)),
            pl.BlockSpec((1, K), lambda i: (0, 0)),
        ],
        out_specs=[
            pl.BlockSpec((1, 1, TILE), lambda i: (i, 0, 0)),
            pl.BlockSpec((1, 1, TILE), lambda i: (i, 0, 0)),
        ],
        out_shape=[
            jax.ShapeDtypeStruct((GRID, 1, TILE), jnp.int32),
            jax.ShapeDtypeStruct((GRID, 1, TILE), jnp.float32),
        ],
    )(x2d, W, x2, w2)


def _sc_gather(W, idx):
    info = plsc.get_sparse_core_info()
    nc, ns = info.num_cores, info.num_subcores
    nw = nc * ns
    b_per_w = N // nw
    ch = 128
    n_ch = b_per_w // ch
    mesh = plsc.VectorSubcoreMesh(core_axis_name="c", subcore_axis_name="s")

    @functools.partial(
        pl.kernel,
        mesh=mesh,
        out_type=jax.ShapeDtypeStruct((N, D), jnp.float32),
        scratch_types=[
            pltpu.VMEM((ch,), jnp.int32),
            pltpu.VMEM((ch, D), jnp.float32),
            pltpu.SemaphoreType.DMA,
        ],
    )
    def gk(table_hbm, idx_hbm, out_hbm, idx_v, rows_v, sem):
        wid = lax.axis_index("s") * nc + lax.axis_index("c")
        for c in range(n_ch):
            base = wid * b_per_w + c * ch
            pltpu.sync_copy(idx_hbm.at[pl.ds(base, ch)], idx_v)
            pltpu.async_copy(table_hbm.at[idx_v], rows_v, sem).wait()
            pltpu.sync_copy(rows_v, out_hbm.at[pl.ds(base, ch)])

    return gk(W, idx)


def kernel(inputs, W):
    x2d = inputs.reshape(-1, D)
    x2 = jnp.sum(x2d ** 2, axis=1, keepdims=True)
    w2 = jnp.sum(W ** 2, axis=1).reshape(1, K)
    idx3, mind3 = _nearest(x2d, W, x2, w2)
    indices = idx3.reshape(N)
    mind = mind3.reshape(N)
    quantized = _sc_gather(W, indices)
    codebook_loss = jnp.sum(mind) / (N * D)
    commitment_loss = codebook_loss * COMMITMENT_COST
    quantized_st = quantized.reshape(inputs.shape)
    return quantized_st, indices, commitment_loss, codebook_loss


# speedup vs baseline: 1.6133x; 1.6133x over previous
import functools

import jax
import jax.numpy as jnp
from jax import lax
from jax.experimental import pallas as pl
from jax.experimental.pallas import tpu as pltpu
from jax.experimental.pallas import tpu_sc as plsc

K = 8192
D = 256
N = 16384
COMMITMENT_COST = 0.25

TILE = 256
GRID = N // TILE


GROUPS = ((0, 2736), (2736, 5472), (5472, 8192))


def _argmin_body(x_ref, wm2_ref, x2_ref, w2_ref, iotaf_ref, idx_ref, mind_ref):
    x = x_ref[...]
    wm2 = wm2_ref[...]
    mm2 = lax.dot_general(x.astype(jnp.bfloat16), wm2,
                          (((1,), (1,)), ((), ())),
                          preferred_element_type=jnp.float32)
    d = (x2_ref[...] + w2_ref[...]) + mm2
    iotaf = iotaf_ref[...]
    ms, js = [], []
    for lo, hi in GROUPS:
        dg = d[:, lo:hi]
        mg = jnp.min(dg, axis=1, keepdims=True)
        jg = jnp.min(jnp.where(dg == mg, iotaf[:, lo:hi], jnp.float32(K)),
                     axis=1, keepdims=True).astype(jnp.int32)
        ms.append(mg)
        js.append(jg)
    a_v, a_i = ms[0], js[0]
    for g in (1, 2):
        a_vb = a_v.astype(jnp.bfloat16).astype(jnp.float32)
        m_g, i_g = ms[g], js[g]
        keep_v = a_vb < m_g
        eq = a_vb == m_g
        a_v = jnp.where(keep_v, a_vb, m_g)
        a_i = jnp.where(keep_v | (eq & (a_i < i_g)), a_i, i_g)
    idx_ref[0, :, :] = a_i
    mind_ref[0, :, :] = jnp.minimum(jnp.minimum(ms[0], ms[1]), ms[2])


def _nearest(x2d, Wm2, x2, w2, iotaf):
    return pl.pallas_call(
        _argmin_body,
        grid=(GRID,),
        in_specs=[
            pl.BlockSpec((TILE, D), lambda i: (i, 0)),
            pl.BlockSpec((K, D), lambda i: (0, 0)),
            pl.BlockSpec((TILE, 1), lambda i: (i, 0)),
            pl.BlockSpec((1, K), lambda i: (0, 0)),
            pl.BlockSpec((1, K), lambda i: (0, 0)),
        ],
        out_specs=[
            pl.BlockSpec((1, TILE, 1), lambda i: (i, 0, 0)),
            pl.BlockSpec((1, TILE, 1), lambda i: (i, 0, 0)),
        ],
        out_shape=[
            jax.ShapeDtypeStruct((GRID, TILE, 1), jnp.int32),
            jax.ShapeDtypeStruct((GRID, TILE, 1), jnp.float32),
        ],
    )(x2d, Wm2, x2, w2, iotaf)


def _sc_gather(W, idx):
    info = plsc.get_sparse_core_info()
    nc, ns = info.num_cores, info.num_subcores
    nw = nc * ns
    b_per_w = N // nw
    ch = 128
    n_ch = b_per_w // ch
    mesh = plsc.VectorSubcoreMesh(core_axis_name="c", subcore_axis_name="s")

    @functools.partial(
        pl.kernel,
        mesh=mesh,
        out_type=jax.ShapeDtypeStruct((N, D), jnp.float32),
        scratch_types=[
            pltpu.VMEM((ch,), jnp.int32),
            pltpu.VMEM((ch, D), jnp.float32),
            pltpu.SemaphoreType.DMA,
        ],
    )
    def gk(table_hbm, idx_hbm, out_hbm, idx_v, rows_v, sem):
        wid = lax.axis_index("s") * nc + lax.axis_index("c")
        for c in range(n_ch):
            base = wid * b_per_w + c * ch
            pltpu.sync_copy(idx_hbm.at[pl.ds(base, ch)], idx_v)
            pltpu.async_copy(table_hbm.at[idx_v], rows_v, sem).wait()
            pltpu.sync_copy(rows_v, out_hbm.at[pl.ds(base, ch)])

    return gk(W, idx)


def kernel(inputs, W):
    x2d = inputs.reshape(-1, D)
    x2 = jnp.sum(x2d ** 2, axis=1, keepdims=True)
    w2 = jnp.sum(W ** 2, axis=1).reshape(1, K)
    iotaf = jnp.arange(K, dtype=jnp.float32).reshape(1, K)
    idx3, mind3 = _nearest(x2d, (-2.0 * W).astype(jnp.bfloat16), x2, w2,
                           iotaf)
    indices = idx3.reshape(N)
    mind = mind3.reshape(N)
    quantized = _sc_gather(W, indices)
    codebook_loss = jnp.sum(mind) / (N * D)
    commitment_loss = codebook_loss * COMMITMENT_COST
    quantized_st = quantized.reshape(inputs.shape)
    return quantized_st, indices, commitment_loss, codebook_loss


# speedup vs baseline: 1.6672x; 1.0334x over previous
import functools

import jax
import jax.numpy as jnp
from jax import lax
from jax.experimental import pallas as pl
from jax.experimental.pallas import tpu as pltpu
from jax.experimental.pallas import tpu_sc as plsc

K = 8192
D = 256
N = 16384
COMMITMENT_COST = 0.25

TILE = 512
GRID = N // TILE


GROUPS = ((0, 2736), (2736, 5472), (5472, 8192))


def _argmin_body(x_ref, wm2_ref, x2_ref, w2_ref, iotaf_ref, idx_ref, mind_ref):
    x = x_ref[...]
    wm2 = wm2_ref[...]
    mm2 = lax.dot_general(x.astype(jnp.bfloat16), wm2,
                          (((1,), (1,)), ((), ())),
                          preferred_element_type=jnp.float32)
    d = (x2_ref[...] + w2_ref[...]) + mm2
    iotaf = iotaf_ref[...]
    ms, js = [], []
    for lo, hi in GROUPS:
        dg = d[:, lo:hi]
        mg = jnp.min(dg, axis=1, keepdims=True)
        jg = jnp.min(jnp.where(dg == mg, iotaf[:, lo:hi], jnp.float32(K)),
                     axis=1, keepdims=True).astype(jnp.int32)
        ms.append(mg)
        js.append(jg)
    a_v, a_i = ms[0], js[0]
    for g in (1, 2):
        a_vb = a_v.astype(jnp.bfloat16).astype(jnp.float32)
        m_g, i_g = ms[g], js[g]
        keep_v = a_vb < m_g
        eq = a_vb == m_g
        a_v = jnp.where(keep_v, a_vb, m_g)
        a_i = jnp.where(keep_v | (eq & (a_i < i_g)), a_i, i_g)
    idx_ref[0, :, :] = a_i
    mind_ref[0, :, :] = jnp.minimum(jnp.minimum(ms[0], ms[1]), ms[2])


def _nearest(x2d, Wm2, x2, w2, iotaf):
    return pl.pallas_call(
        _argmin_body,
        grid=(GRID,),
        in_specs=[
            pl.BlockSpec((TILE, D), lambda i: (i, 0)),
            pl.BlockSpec((K, D), lambda i: (0, 0)),
            pl.BlockSpec((TILE, 1), lambda i: (i, 0)),
            pl.BlockSpec((1, K), lambda i: (0, 0)),
            pl.BlockSpec((1, K), lambda i: (0, 0)),
        ],
        out_specs=[
            pl.BlockSpec((1, TILE, 1), lambda i: (i, 0, 0)),
            pl.BlockSpec((1, TILE, 1), lambda i: (i, 0, 0)),
        ],
        out_shape=[
            jax.ShapeDtypeStruct((GRID, TILE, 1), jnp.int32),
            jax.ShapeDtypeStruct((GRID, TILE, 1), jnp.float32),
        ],
    )(x2d, Wm2, x2, w2, iotaf)


def _sc_gather(W, idx):
    info = plsc.get_sparse_core_info()
    nc, ns = info.num_cores, info.num_subcores
    nw = nc * ns
    b_per_w = N // nw
    ch = 128
    n_ch = b_per_w // ch
    mesh = plsc.VectorSubcoreMesh(core_axis_name="c", subcore_axis_name="s")

    @functools.partial(
        pl.kernel,
        mesh=mesh,
        out_type=jax.ShapeDtypeStruct((N, D), jnp.float32),
        scratch_types=[
            pltpu.VMEM((ch,), jnp.int32),
            pltpu.VMEM((ch, D), jnp.float32),
            pltpu.SemaphoreType.DMA,
        ],
    )
    def gk(table_hbm, idx_hbm, out_hbm, idx_v, rows_v, sem):
        wid = lax.axis_index("s") * nc + lax.axis_index("c")
        for c in range(n_ch):
            base = wid * b_per_w + c * ch
            pltpu.sync_copy(idx_hbm.at[pl.ds(base, ch)], idx_v)
            pltpu.async_copy(table_hbm.at[idx_v], rows_v, sem).wait()
            pltpu.sync_copy(rows_v, out_hbm.at[pl.ds(base, ch)])

    return gk(W, idx)


def kernel(inputs, W):
    x2d = inputs.reshape(-1, D)
    x2 = jnp.sum(x2d ** 2, axis=1, keepdims=True)
    w2 = jnp.sum(W ** 2, axis=1).reshape(1, K)
    iotaf = jnp.arange(K, dtype=jnp.float32).reshape(1, K)
    idx3, mind3 = _nearest(x2d, (-2.0 * W).astype(jnp.bfloat16), x2, w2,
                           iotaf)
    indices = idx3.reshape(N)
    mind = mind3.reshape(N)
    quantized = _sc_gather(W, indices)
    codebook_loss = jnp.sum(mind) / (N * D)
    commitment_loss = codebook_loss * COMMITMENT_COST
    quantized_st = quantized.reshape(inputs.shape)
    return quantized_st, indices, commitment_loss, codebook_loss


# speedup vs baseline: 1.7555x; 1.0529x over previous
import functools

import jax
import jax.numpy as jnp
from jax import lax
from jax.experimental import pallas as pl
from jax.experimental.pallas import tpu as pltpu
from jax.experimental.pallas import tpu_sc as plsc

K = 8192
D = 256
N = 16384
COMMITMENT_COST = 0.25

TILE = 512
GRID = N // TILE


GROUPS = ((0, 2736), (2736, 5472), (5472, 8192))


def _argmin_body(x_ref, wm2_ref, w2_ref, iotaf_ref, idx_ref, loss_ref):
    x = x_ref[...]
    wm2 = wm2_ref[...]
    x2 = jnp.sum(x * x, axis=1, keepdims=True)
    mm2 = lax.dot_general(x.astype(jnp.bfloat16), wm2,
                          (((1,), (1,)), ((), ())),
                          preferred_element_type=jnp.float32)
    d = (x2 + w2_ref[...]) + mm2
    iotaf = iotaf_ref[...]
    ms, js = [], []
    for lo, hi in GROUPS:
        dg = d[:, lo:hi]
        mg = jnp.min(dg, axis=1, keepdims=True)
        jg = jnp.min(jnp.where(dg == mg, iotaf[:, lo:hi], jnp.float32(K)),
                     axis=1, keepdims=True).astype(jnp.int32)
        ms.append(mg)
        js.append(jg)
    a_v, a_i = ms[0], js[0]
    for g in (1, 2):
        a_vb = a_v.astype(jnp.bfloat16).astype(jnp.float32)
        m_g, i_g = ms[g], js[g]
        keep_v = a_vb < m_g
        eq = a_vb == m_g
        a_v = jnp.where(keep_v, a_vb, m_g)
        a_i = jnp.where(keep_v | (eq & (a_i < i_g)), a_i, i_g)
    idx_ref[0, :, :] = a_i
    tile_sum = jnp.sum(jnp.minimum(jnp.minimum(ms[0], ms[1]), ms[2]))

    @pl.when(pl.program_id(0) == 0)
    def _():
        loss_ref[0, 0] = 0.0

    loss_ref[0, 0] += tile_sum


def _nearest(x2d, Wm2, w2, iotaf):
    return pl.pallas_call(
        _argmin_body,
        grid=(GRID,),
        in_specs=[
            pl.BlockSpec((TILE, D), lambda i: (i, 0)),
            pl.BlockSpec((K, D), lambda i: (0, 0)),
            pl.BlockSpec((1, K), lambda i: (0, 0)),
            pl.BlockSpec((1, K), lambda i: (0, 0)),
        ],
        out_specs=[
            pl.BlockSpec((1, TILE, 1), lambda i: (i, 0, 0)),
            pl.BlockSpec((1, 1), lambda i: (0, 0),
                         memory_space=pltpu.SMEM),
        ],
        out_shape=[
            jax.ShapeDtypeStruct((GRID, TILE, 1), jnp.int32),
            jax.ShapeDtypeStruct((1, 1), jnp.float32),
        ],
    )(x2d, Wm2, w2, iotaf)


def _sc_gather(W, idx):
    info = plsc.get_sparse_core_info()
    nc, ns = info.num_cores, info.num_subcores
    nw = nc * ns
    b_per_w = N // nw
    ch = 128
    n_ch = b_per_w // ch
    mesh = plsc.VectorSubcoreMesh(core_axis_name="c", subcore_axis_name="s")

    @functools.partial(
        pl.kernel,
        mesh=mesh,
        out_type=jax.ShapeDtypeStruct((N, D), jnp.float32),
        scratch_types=[
            pltpu.VMEM((ch,), jnp.int32),
            pltpu.VMEM((ch, D), jnp.float32),
            pltpu.SemaphoreType.DMA,
        ],
    )
    def gk(table_hbm, idx_hbm, out_hbm, idx_v, rows_v, sem):
        wid = lax.axis_index("s") * nc + lax.axis_index("c")
        for c in range(n_ch):
            base = wid * b_per_w + c * ch
            pltpu.sync_copy(idx_hbm.at[pl.ds(base, ch)], idx_v)
            pltpu.async_copy(table_hbm.at[idx_v], rows_v, sem).wait()
            pltpu.sync_copy(rows_v, out_hbm.at[pl.ds(base, ch)])

    return gk(W, idx)


def kernel(inputs, W):
    x2d = inputs.reshape(-1, D)
    w2 = jnp.sum(W ** 2, axis=1).reshape(1, K)
    iotaf = jnp.arange(K, dtype=jnp.float32).reshape(1, K)
    idx3, loss = _nearest(x2d, (-2.0 * W).astype(jnp.bfloat16), w2, iotaf)
    indices = idx3.reshape(N)
    quantized = _sc_gather(W, indices)
    codebook_loss = loss[0, 0] / (N * D)
    commitment_loss = codebook_loss * COMMITMENT_COST
    quantized_st = quantized.reshape(inputs.shape)
    return quantized_st, indices, commitment_loss, codebook_loss
